# baseline (speedup 1.0000x reference)
import functools

import jax
import jax.numpy as jnp
from jax import lax
from jax.experimental import pallas as pl
from jax.experimental.pallas import tpu as pltpu
from jax.experimental.pallas import tpu_sc as plsc

NC = 2
NS = 16
LANES = 16


def _pick_row_block(n, target=1024):
    for r in range(min(n, target), 7, -8):
        if n % r == 0:
            return r
    return n


def _tc_pre_node(x, W1n, W1s):
    N, D = x.shape
    DH = D // 2
    R = _pick_row_block(N)
    NB = N // R

    def body(x_ref, w1n_ref, w1s_ref, t_ref, s_ref):
        a = x_ref[...]
        t_ref[...] = jnp.dot(a, w1n_ref[...], preferred_element_type=jnp.float32,
                    precision=jax.lax.Precision.HIGHEST)
        s_ref[...] = jnp.dot(a, w1s_ref[...], preferred_element_type=jnp.float32,
                    precision=jax.lax.Precision.HIGHEST)

    return pl.pallas_call(
        body,
        grid=(NB, 2),
        in_specs=[
            pl.BlockSpec((R, D), lambda i, j: (i, 0)),
            pl.BlockSpec((D, DH), lambda i, j: (0, j)),
            pl.BlockSpec((D, DH), lambda i, j: (0, j)),
        ],
        out_specs=[
            pl.BlockSpec((R, DH), lambda i, j: (j * NB + i, 0)),
            pl.BlockSpec((R, DH), lambda i, j: (i, j)),
        ],
        out_shape=[
            jax.ShapeDtypeStruct((2 * N, DH), jnp.float32),
            jax.ShapeDtypeStruct((N, D), jnp.float32),
        ],
    )(x, W1n, W1s)


def _tc_pre_edge(ea, W1e, W2e):
    E, DE = ea.shape
    D = W1e.shape[1]
    DH = D // 2
    R = _pick_row_block(E, 8000)
    EB = E // R

    def body(ea_ref, w1_ref, w2_ref, o1_ref, o2_ref):
        a = ea_ref[...]
        o1_ref[...] = jnp.dot(a, w1_ref[...], preferred_element_type=jnp.float32,
                    precision=jax.lax.Precision.HIGHEST)
        o2_ref[...] = jnp.dot(a, w2_ref[...], preferred_element_type=jnp.float32,
                    precision=jax.lax.Precision.HIGHEST)

    return pl.pallas_call(
        body,
        grid=(EB, 2),
        in_specs=[
            pl.BlockSpec((R, DE), lambda i, j: (i, 0)),
            pl.BlockSpec((DE, DH), lambda i, j: (0, j)),
            pl.BlockSpec((DE, DH), lambda i, j: (0, j)),
        ],
        out_specs=[
            pl.BlockSpec((R, DH), lambda i, j: (j * EB + i, 0)),
            pl.BlockSpec((R, DH), lambda i, j: (j * EB + i, 0)),
        ],
        out_shape=[
            jax.ShapeDtypeStruct((2 * E, DH), jnp.float32),
            jax.ShapeDtypeStruct((2 * E, DH), jnp.float32),
        ],
    )(ea, W1e, W2e)


def _tc_mid(agg, skip, b, Wn, Ws):
    N, D = skip.shape
    DH = D // 2
    R = _pick_row_block(N)
    NB = N // R

    def body(lo_ref, hi_ref, skip_ref, b_ref, wn_ref, ws_ref, t_ref, s_ref):
        h = jnp.concatenate([lo_ref[...], hi_ref[...]], axis=1)
        h = jnp.maximum(h + skip_ref[...] + b_ref[...], 0.0)
        t_ref[...] = jnp.dot(h, wn_ref[...], preferred_element_type=jnp.float32,
                    precision=jax.lax.Precision.HIGHEST)
        s_ref[...] = jnp.dot(h, ws_ref[...], preferred_element_type=jnp.float32,
                    precision=jax.lax.Precision.HIGHEST)

    return pl.pallas_call(
        body,
        grid=(NB, 2),
        in_specs=[
            pl.BlockSpec((R, DH), lambda i, j: (i, 0)),
            pl.BlockSpec((R, DH), lambda i, j: (NB + i, 0)),
            pl.BlockSpec((R, D), lambda i, j: (i, 0)),
            pl.BlockSpec((1, D), lambda i, j: (0, 0)),
            pl.BlockSpec((D, DH), lambda i, j: (0, j)),
            pl.BlockSpec((D, DH), lambda i, j: (0, j)),
        ],
        out_specs=[
            pl.BlockSpec((R, DH), lambda i, j: (j * NB + i, 0)),
            pl.BlockSpec((R, DH), lambda i, j: (i, j)),
        ],
        out_shape=[
            jax.ShapeDtypeStruct((2 * N, DH), jnp.float32),
            jax.ShapeDtypeStruct((N, D), jnp.float32),
        ],
    )(agg, agg, skip, b, Wn, Ws)


def _tc_emb(agg, skip, b, We1, be1):
    N, D = skip.shape
    DH = D // 2
    R = _pick_row_block(N)
    NB = N // R

    def body(lo_ref, hi_ref, skip_ref, b_ref, wa_ref, wb_ref, be1_ref, a_ref, b2_ref):
        h = jnp.concatenate([lo_ref[...], hi_ref[...]], axis=1)
        h = jnp.maximum(h + skip_ref[...] + b_ref[...], 0.0)
        a_ref[...] = (
            jnp.dot(h, wa_ref[...], preferred_element_type=jnp.float32,
                    precision=jax.lax.Precision.HIGHEST) + be1_ref[...]
        )
        b2_ref[...] = jnp.dot(h, wb_ref[...], preferred_element_type=jnp.float32,
                    precision=jax.lax.Precision.HIGHEST)

    return pl.pallas_call(
        body,
        grid=(NB, 2),
        in_specs=[
            pl.BlockSpec((R, DH), lambda i, j: (i, 0)),
            pl.BlockSpec((R, DH), lambda i, j: (NB + i, 0)),
            pl.BlockSpec((R, D), lambda i, j: (i, 0)),
            pl.BlockSpec((1, D), lambda i, j: (0, 0)),
            pl.BlockSpec((D, DH), lambda i, j: (0, j)),
            pl.BlockSpec((D, DH), lambda i, j: (1, j)),
            pl.BlockSpec((1, DH), lambda i, j: (0, j)),
        ],
        out_specs=[
            pl.BlockSpec((R, DH), lambda i, j: (i, j)),
            pl.BlockSpec((R, DH), lambda i, j: (i, j)),
        ],
        out_shape=[
            jax.ShapeDtypeStruct((N, D), jnp.float32),
            jax.ShapeDtypeStruct((N, D), jnp.float32),
        ],
    )(agg, agg, skip, b, We1, We1, be1)


def _tc_final(agg, skip, b):
    N, D = skip.shape
    DH = D // 2
    R = _pick_row_block(N)
    NB = N // R

    def body(agg_ref, skip_ref, b_ref, o_ref):
        o_ref[...] = jnp.maximum(agg_ref[...] + skip_ref[...] + b_ref[...], 0.0)

    return pl.pallas_call(
        body,
        grid=(NB, 2),
        in_specs=[
            pl.BlockSpec((R, DH), lambda i, j: (j * NB + i, 0)),
            pl.BlockSpec((R, DH), lambda i, j: (i, j)),
            pl.BlockSpec((1, DH), lambda i, j: (0, j)),
        ],
        out_specs=pl.BlockSpec((R, DH), lambda i, j: (i, j)),
        out_shape=jax.ShapeDtypeStruct((N, D), jnp.float32),
    )(agg, skip, b)


def _tc_logits(s16, be2):
    E, L = s16.shape
    R = _pick_row_block(E, 8000)
    EB = E // R

    def body(s_ref, b_ref, lo_ref, at_ref):
        v = jnp.sum(s_ref[...], axis=1, keepdims=True) + b_ref[...]
        lo_ref[...] = v
        at_ref[...] = jax.nn.sigmoid(v)

    return pl.pallas_call(
        body,
        grid=(EB,),
        in_specs=[
            pl.BlockSpec((R, L), lambda i: (i, 0)),
            pl.BlockSpec((1, 1), lambda i: (0, 0)),
        ],
        out_specs=[
            pl.BlockSpec((R, 1), lambda i: (i, 0)),
            pl.BlockSpec((R, 1), lambda i: (i, 0)),
        ],
        out_shape=[
            jax.ShapeDtypeStruct((E, 1), jnp.float32),
            jax.ShapeDtypeStruct((E, 1), jnp.float32),
        ],
    )(s16, be2)


def _pick_chunk(n, cap=128, mult=8):
    for k in range(cap - cap % mult, mult - 1, -mult):
        if n % k == 0:
            return k
    return mult


def _pick_writers(n):
    for wt in range(NS, 0, -1):
        if n % wt == 0 and (n // wt) % 8 == 0:
            return wt, n // wt
    return 1, n


def _sc_msgpass(src, dst, table, ew, att, store_m):
    E = src.shape[0]
    twoN, DH = table.shape
    N = twoN // 2
    EPT = E // NS
    K = _pick_chunk(EPT, mult=LANES)
    NCH = EPT // K
    WT, RPT = _pick_writers(N)
    use_att = att is not None

    mesh = plsc.VectorSubcoreMesh(core_axis_name="c", subcore_axis_name="s")

    out_type = [jax.ShapeDtypeStruct((2 * N, DH), jnp.float32)]
    if store_m:
        out_type = [jax.ShapeDtypeStruct((2 * E, DH), jnp.float32)] + out_type

    scratch = [
        pltpu.VMEM((K,), jnp.int32),
        pltpu.VMEM((K,), jnp.int32),
        pltpu.VMEM((K, DH), jnp.float32),
        pltpu.VMEM((K, DH), jnp.float32),
        pltpu.VMEM((K,), jnp.float32),
        pltpu.VMEM_SHARED((N, DH), jnp.float32),
        pltpu.SemaphoreType.DMA,
    ]

    def body(*refs):
        i = 0
        src_hbm = refs[i]; i += 1
        dst_hbm = refs[i]; i += 1
        table_hbm = refs[i]; i += 1
        ew_hbm = refs[i]; i += 1
        if use_att:
            att_hbm = refs[i]; i += 1
        z_hbm = refs[i]; i += 1
        if store_m:
            m_hbm = refs[i]; i += 1
        agg_hbm = refs[i]; i += 1
        idx_v, dst_v, rows_v, ew_v, att_v, acc, sem = refs[i:]

        cid = lax.axis_index("c")
        sid = lax.axis_index("s")

        @pl.when(sid < WT)
        def _():
            pltpu.sync_copy(z_hbm, acc.at[pl.ds(sid * RPT, RPT)])

        plsc.subcore_barrier()

        tile_base = sid * EPT
        row_off = cid * N
        e_off = cid * E

        def chunk(g, carry):
            base = tile_base + g * K
            pltpu.sync_copy(src_hbm.at[pl.ds(base, K)], idx_v)
            pltpu.sync_copy(dst_hbm.at[pl.ds(base, K)], dst_v)
            for j in range(K // LANES):
                sl = pl.ds(j * LANES, LANES)
                idx_v[sl] = idx_v[sl] + row_off
            pltpu.async_copy(table_hbm.at[idx_v], rows_v, sem).wait()
            pltpu.sync_copy(ew_hbm.at[pl.ds(e_off + base, K)], ew_v)
            if use_att:
                pltpu.sync_copy(att_hbm.at[pl.ds(base, K)], att_v)

            def group(g2, c2):
                e0 = g2 * LANES
                if use_att:
                    att16 = att_v[pl.ds(e0, LANES)]
                for l in range(LANES):
                    e = e0 + l
                    for j in range(DH // LANES):
                        sl = pl.ds(j * LANES, LANES)
                        v = jnp.maximum(rows_v[e, sl] + ew_v[e, sl], 0.0)
                        if use_att:
                            v = v * att16[l]
                        rows_v[e, sl] = v
                return c2

            lax.fori_loop(0, K // LANES, group, 0)
            if store_m:
                pltpu.sync_copy(rows_v, m_hbm.at[pl.ds(e_off + base, K)])
            pltpu.sync_copy(rows_v, acc.at[dst_v], add=True)
            return carry

        lax.fori_loop(0, NCH, chunk, 0)
        plsc.subcore_barrier()

        @pl.when(sid < WT)
        def _():
            pltpu.sync_copy(
                acc.at[pl.ds(sid * RPT, RPT)],
                agg_hbm.at[pl.ds(row_off + sid * RPT, RPT)],
            )

    zrows = jnp.zeros((RPT, DH), jnp.float32)
    args = [src, dst, table, ew]
    if use_att:
        args.append(att)
    args.append(zrows)

    out = pl.kernel(body, out_type=out_type, mesh=mesh, scratch_types=scratch)(*args)
    return tuple(out) if store_m else out[0]


def _sc_scale_agg(m, att, dst, N):
    twoE, DH = m.shape
    E = twoE // 2
    EPT = E // NS
    K = _pick_chunk(EPT, mult=LANES)
    NCH = EPT // K
    WT, RPT = _pick_writers(N)

    mesh = plsc.VectorSubcoreMesh(core_axis_name="c", subcore_axis_name="s")

    scratch = [
        pltpu.VMEM((K,), jnp.int32),
        pltpu.VMEM((K, DH), jnp.float32),
        pltpu.VMEM((K,), jnp.float32),
        pltpu.VMEM_SHARED((N, DH), jnp.float32),
        pltpu.SemaphoreType.DMA,
    ]

    def body(m_hbm, att_hbm, dst_hbm, z_hbm, agg_hbm, dst_v, rows_v, att_v, acc, sem):
        cid = lax.axis_index("c")
        sid = lax.axis_index("s")

        @pl.when(sid < WT)
        def _():
            pltpu.sync_copy(z_hbm, acc.at[pl.ds(sid * RPT, RPT)])

        plsc.subcore_barrier()
        tile_base = sid * EPT
        e_off = cid * E

        def chunk(g, carry):
            base = tile_base + g * K
            pltpu.sync_copy(dst_hbm.at[pl.ds(base, K)], dst_v)
            pltpu.sync_copy(m_hbm.at[pl.ds(e_off + base, K)], rows_v)
            pltpu.sync_copy(att_hbm.at[pl.ds(base, K)], att_v)

            def group(g2, c2):
                e0 = g2 * LANES
                att16 = att_v[pl.ds(e0, LANES)]
                for l in range(LANES):
                    e = e0 + l
                    for j in range(DH // LANES):
                        sl = pl.ds(j * LANES, LANES)
                        rows_v[e, sl] = rows_v[e, sl] * att16[l]
                return c2

            lax.fori_loop(0, K // LANES, group, 0)
            pltpu.sync_copy(rows_v, acc.at[dst_v], add=True)
            return carry

        lax.fori_loop(0, NCH, chunk, 0)
        plsc.subcore_barrier()

        @pl.when(sid < WT)
        def _():
            pltpu.sync_copy(
                acc.at[pl.ds(sid * RPT, RPT)],
                agg_hbm.at[pl.ds(cid * N + sid * RPT, RPT)],
            )

    zrows = jnp.zeros((RPT, DH), jnp.float32)
    out = pl.kernel(
        body,
        out_type=[jax.ShapeDtypeStruct((2 * N, DH), jnp.float32)],
        mesh=mesh,
        scratch_types=scratch,
    )(m, att, dst, zrows)
    return out[0]


def _sc_att(src, dst, embA, embB, we2):
    E = src.shape[0]
    N, D = embA.shape
    NW = NC * NS
    EPT = E // NW
    K = _pick_chunk(EPT, 64)
    NCH = EPT // K

    mesh = plsc.VectorSubcoreMesh(core_axis_name="c", subcore_axis_name="s")

    scratch = [
        pltpu.VMEM((K,), jnp.int32),
        pltpu.VMEM((K,), jnp.int32),
        pltpu.VMEM((K, D), jnp.float32),
        pltpu.VMEM((K, D), jnp.float32),
        pltpu.VMEM((K, LANES), jnp.float32),
        pltpu.VMEM((D,), jnp.float32),
        pltpu.SemaphoreType.DMA,
    ]

    def body(src_hbm, dst_hbm, a_hbm, b_hbm, w_hbm, s_hbm,
             sidx, didx, a_v, b_v, o_v, w_v, sem):
        cid = lax.axis_index("c")
        sid = lax.axis_index("s")
        wid = sid * NC + cid
        pltpu.sync_copy(w_hbm, w_v)
        tile_base = wid * EPT

        def chunk(g, carry):
            base = tile_base + g * K
            pltpu.sync_copy(src_hbm.at[pl.ds(base, K)], sidx)
            pltpu.sync_copy(dst_hbm.at[pl.ds(base, K)], didx)
            pltpu.async_copy(a_hbm.at[sidx], a_v, sem).wait()
            pltpu.async_copy(b_hbm.at[didx], b_v, sem).wait()

            def edge(e, c2):
                acc = jnp.zeros((LANES,), jnp.float32)
                for j in range(D // LANES):
                    sl = pl.ds(j * LANES, LANES)
                    t = jnp.maximum(a_v[e, sl] + b_v[e, sl], 0.0)
                    acc = acc + t * w_v[sl]
                o_v[e, pl.ds(0, LANES)] = acc
                return c2

            lax.fori_loop(0, K, edge, 0)
            pltpu.sync_copy(o_v, s_hbm.at[pl.ds(base, K)])
            return carry

        lax.fori_loop(0, NCH, chunk, 0)

    out = pl.kernel(
        body,
        out_type=[jax.ShapeDtypeStruct((E, LANES), jnp.float32)],
        mesh=mesh,
        scratch_types=scratch,
    )(src, dst, embA, embB, we2)
    return out[0]


def kernel(x, edge_index, edge_attr, batch, W1n, W1e, W1s, b1,
           W2n, W2e, W2s, b2, We1, be1, We2, be2):
    N, D = x.shape
    E = edge_index.shape[1]
    src = edge_index[0]
    dst = edge_index[1]
    b1r = b1.reshape(1, D)
    b2r = b2.reshape(1, D)
    be1r = be1.reshape(1, D)
    we2v = We2.reshape(D)
    be2r = be2.reshape(1, 1)

    xW1n_t, xW1s = _tc_pre_node(x, W1n, W1s)
    eW1_t, eW2_t = _tc_pre_edge(edge_attr, W1e, W2e)

    m1, agg1 = _sc_msgpass(src, dst, xW1n_t, eW1_t, None, store_m=True)
    h1W2n_t, h1W2s = _tc_mid(agg1, xW1s, b1r, W2n, W2s)
    agg2 = _sc_msgpass(src, dst, h1W2n_t, eW2_t, None, store_m=False)
    embA, embB = _tc_emb(agg2, h1W2s, b2r, We1, be1r)
    s16 = _sc_att(src, dst, embA, embB, we2v)
    logits, att2 = _tc_logits(s16, be2r)
    att = att2.reshape(E)

    agg1p = _sc_scale_agg(m1, att, dst, N)
    h1pW2n_t, h1pW2s = _tc_mid(agg1p, xW1s, b1r, W2n, W2s)
    agg2p = _sc_msgpass(src, dst, h1pW2n_t, eW2_t, att, store_m=False)
    node_embeddings = _tc_final(agg2p, h1pW2s, b2r)

    return (logits, att2, node_embeddings)


# speedup vs baseline: 1.6847x; 1.6847x over previous
import functools

import jax
import jax.numpy as jnp
from jax import lax
from jax.experimental import pallas as pl
from jax.experimental.pallas import tpu as pltpu
from jax.experimental.pallas import tpu_sc as plsc

NC = 2
NS = 16
LANES = 16


def _pick_row_block(n, target=1024):
    for r in range(min(n, target), 7, -8):
        if n % r == 0:
            return r
    return n


def _dot(a, b):
    return jnp.dot(a, b, preferred_element_type=jnp.float32,
                   precision=jax.lax.Precision.HIGHEST)


def _tc_pre_node(x, W1n, W1s):
    N, D = x.shape
    DH = D // 2
    R = _pick_row_block(N)
    NB = N // R

    def body(x_ref, w1n_ref, w1s_ref, t_ref, s_ref):
        a = x_ref[...]
        t_ref[...] = _dot(a, w1n_ref[...])
        s_ref[...] = _dot(a, w1s_ref[...])

    return pl.pallas_call(
        body,
        grid=(NB, 2),
        in_specs=[
            pl.BlockSpec((R, D), lambda i, j: (i, 0)),
            pl.BlockSpec((D, DH), lambda i, j: (0, j)),
            pl.BlockSpec((D, DH), lambda i, j: (0, j)),
        ],
        out_specs=[
            pl.BlockSpec((R, DH), lambda i, j: (j * NB + i, 0)),
            pl.BlockSpec((R, DH), lambda i, j: (i, j)),
        ],
        out_shape=[
            jax.ShapeDtypeStruct((2 * N, DH), jnp.float32),
            jax.ShapeDtypeStruct((N, D), jnp.float32),
        ],
    )(x, W1n, W1s)


def _tc_pre_edge(ea, W1e, W2e):
    E, DE = ea.shape
    D = W1e.shape[1]
    DH = D // 2
    R = _pick_row_block(E, 8000)
    EB = E // R

    def body(ea_ref, w1_ref, w2_ref, o1_ref, o2_ref):
        a = ea_ref[...]
        o1_ref[...] = _dot(a, w1_ref[...])
        o2_ref[...] = _dot(a, w2_ref[...])

    return pl.pallas_call(
        body,
        grid=(EB, 2),
        in_specs=[
            pl.BlockSpec((R, DE), lambda i, j: (i, 0)),
            pl.BlockSpec((DE, DH), lambda i, j: (0, j)),
            pl.BlockSpec((DE, DH), lambda i, j: (0, j)),
        ],
        out_specs=[
            pl.BlockSpec((R, DH), lambda i, j: (j * EB + i, 0)),
            pl.BlockSpec((R, DH), lambda i, j: (j * EB + i, 0)),
        ],
        out_shape=[
            jax.ShapeDtypeStruct((2 * E, DH), jnp.float32),
            jax.ShapeDtypeStruct((2 * E, DH), jnp.float32),
        ],
    )(ea, W1e, W2e)


def _tc_mid(agg, skip, b, Wn, Ws):
    N, D = skip.shape
    DH = D // 2
    R = _pick_row_block(N)
    NB = N // R

    def body(lo_ref, hi_ref, skip_ref, b_ref, wn_ref, ws_ref, t_ref, s_ref):
        h = jnp.concatenate([lo_ref[...], hi_ref[...]], axis=1)
        h = jnp.maximum(h + skip_ref[...] + b_ref[...], 0.0)
        t_ref[...] = _dot(h, wn_ref[...])
        s_ref[...] = _dot(h, ws_ref[...])

    return pl.pallas_call(
        body,
        grid=(NB, 2),
        in_specs=[
            pl.BlockSpec((R, DH), lambda i, j: (i, 0)),
            pl.BlockSpec((R, DH), lambda i, j: (NB + i, 0)),
            pl.BlockSpec((R, D), lambda i, j: (i, 0)),
            pl.BlockSpec((1, D), lambda i, j: (0, 0)),
            pl.BlockSpec((D, DH), lambda i, j: (0, j)),
            pl.BlockSpec((D, DH), lambda i, j: (0, j)),
        ],
        out_specs=[
            pl.BlockSpec((R, DH), lambda i, j: (j * NB + i, 0)),
            pl.BlockSpec((R, DH), lambda i, j: (i, j)),
        ],
        out_shape=[
            jax.ShapeDtypeStruct((2 * N, DH), jnp.float32),
            jax.ShapeDtypeStruct((N, D), jnp.float32),
        ],
    )(agg, agg, skip, b, Wn, Ws)


def _tc_emb(agg, skip, b, We1, be1):
    N, D = skip.shape
    DH = D // 2
    R = _pick_row_block(N)
    NB = N // R

    def body(lo_ref, hi_ref, skip_ref, b_ref, wa_ref, wb_ref, be1_ref, a_ref, b2_ref):
        h = jnp.concatenate([lo_ref[...], hi_ref[...]], axis=1)
        h = jnp.maximum(h + skip_ref[...] + b_ref[...], 0.0)
        a_ref[...] = _dot(h, wa_ref[...]) + be1_ref[...]
        b2_ref[...] = _dot(h, wb_ref[...])

    return pl.pallas_call(
        body,
        grid=(NB, 2),
        in_specs=[
            pl.BlockSpec((R, DH), lambda i, j: (i, 0)),
            pl.BlockSpec((R, DH), lambda i, j: (NB + i, 0)),
            pl.BlockSpec((R, D), lambda i, j: (i, 0)),
            pl.BlockSpec((1, D), lambda i, j: (0, 0)),
            pl.BlockSpec((D, DH), lambda i, j: (0, j)),
            pl.BlockSpec((D, DH), lambda i, j: (1, j)),
            pl.BlockSpec((1, DH), lambda i, j: (0, j)),
        ],
        out_specs=[
            pl.BlockSpec((R, DH), lambda i, j: (i, j)),
            pl.BlockSpec((R, DH), lambda i, j: (i, j)),
        ],
        out_shape=[
            jax.ShapeDtypeStruct((N, D), jnp.float32),
            jax.ShapeDtypeStruct((N, D), jnp.float32),
        ],
    )(agg, agg, skip, b, We1, We1, be1)


def _tc_final(agg, skip, b):
    N, D = skip.shape
    DH = D // 2
    R = _pick_row_block(N)
    NB = N // R

    def body(agg_ref, skip_ref, b_ref, o_ref):
        o_ref[...] = jnp.maximum(agg_ref[...] + skip_ref[...] + b_ref[...], 0.0)

    return pl.pallas_call(
        body,
        grid=(NB, 2),
        in_specs=[
            pl.BlockSpec((R, DH), lambda i, j: (j * NB + i, 0)),
            pl.BlockSpec((R, DH), lambda i, j: (i, j)),
            pl.BlockSpec((1, DH), lambda i, j: (0, j)),
        ],
        out_specs=pl.BlockSpec((R, DH), lambda i, j: (i, j)),
        out_shape=jax.ShapeDtypeStruct((N, D), jnp.float32),
    )(agg, skip, b)


def _tc_logits(s16, be2):
    E, L = s16.shape
    R = _pick_row_block(E, 8000)
    EB = E // R

    def body(s_ref, b_ref, lo_ref, at_ref):
        v = jnp.sum(s_ref[...], axis=1, keepdims=True) + b_ref[...]
        lo_ref[...] = v
        at_ref[...] = jax.nn.sigmoid(v)

    return pl.pallas_call(
        body,
        grid=(EB,),
        in_specs=[
            pl.BlockSpec((R, L), lambda i: (i, 0)),
            pl.BlockSpec((1, 1), lambda i: (0, 0)),
        ],
        out_specs=[
            pl.BlockSpec((R, 1), lambda i: (i, 0)),
            pl.BlockSpec((R, 1), lambda i: (i, 0)),
        ],
        out_shape=[
            jax.ShapeDtypeStruct((E, 1), jnp.float32),
            jax.ShapeDtypeStruct((E, 1), jnp.float32),
        ],
    )(s16, be2)


def _pick_chunk(n, cap=128, mult=8):
    for k in range(cap - cap % mult, mult - 1, -mult):
        if n % k == 0:
            return k
    return mult


def _pick_writers(n):
    for wt in range(NS, 0, -1):
        if n % wt == 0 and (n // wt) % 8 == 0:
            return wt, n // wt
    return 1, n


def _sc_msgpass(srcN, dst, table, ew, att, store_m, N, E):
    _, DH = table.shape
    EPT = E // NS
    K = _pick_chunk(EPT, mult=LANES)
    NCH = EPT // K
    WT, RPT = _pick_writers(N)
    use_att = att is not None
    PAIRS = (NCH + 1) // 2

    mesh = plsc.VectorSubcoreMesh(core_axis_name="c", subcore_axis_name="s")

    out_type = [jax.ShapeDtypeStruct((2 * N, DH), jnp.float32)]
    if store_m:
        out_type = [jax.ShapeDtypeStruct((2 * E, DH), jnp.float32)] + out_type

    scratch = [
        pltpu.VMEM((K,), jnp.int32),
        pltpu.VMEM((K,), jnp.int32),
        pltpu.VMEM((K,), jnp.int32),
        pltpu.VMEM((K,), jnp.int32),
        pltpu.VMEM((K,), jnp.float32),
        pltpu.VMEM((K,), jnp.float32),
        pltpu.VMEM((K, DH), jnp.float32),
        pltpu.VMEM((K, DH), jnp.float32),
        pltpu.VMEM((K, DH), jnp.float32),
        pltpu.VMEM_SHARED((N, DH), jnp.float32),
        pltpu.SemaphoreType.DMA,
        pltpu.SemaphoreType.DMA,
        pltpu.SemaphoreType.DMA,
        pltpu.SemaphoreType.DMA,
        pltpu.SemaphoreType.DMA,
    ]

    def body(*refs):
        i = 0
        srcN_hbm = refs[i]; i += 1
        dst_hbm = refs[i]; i += 1
        table_hbm = refs[i]; i += 1
        ew_hbm = refs[i]; i += 1
        if use_att:
            att_hbm = refs[i]; i += 1
        z_hbm = refs[i]; i += 1
        if store_m:
            m_hbm = refs[i]; i += 1
        agg_hbm = refs[i]; i += 1
        (idx_a, idx_b, dst_a, dst_b, att_a, att_b,
         rows_a, rows_b, ew_s, acc,
         sem_ra, sem_rb, sem_e, sem_ma, sem_mb) = refs[i:]

        cid = lax.axis_index("c")
        sid = lax.axis_index("s")

        @pl.when(sid < WT)
        def _():
            pltpu.sync_copy(z_hbm, acc.at[pl.ds(sid * RPT, RPT)])

        plsc.subcore_barrier()

        tile_base = sid * EPT
        row_off = cid * N
        e_off = cid * E

        def issue_idx(g, idx_v, dst_v, att_v, sem_i):
            pltpu.async_copy(
                srcN_hbm.at[pl.ds(e_off + tile_base + g * K, K)], idx_v, sem_i)
            pltpu.async_copy(
                dst_hbm.at[pl.ds(tile_base + g * K, K)], dst_v, sem_i)
            if use_att:
                pltpu.async_copy(
                    att_hbm.at[pl.ds(tile_base + g * K, K)], att_v, sem_i)

        def wait_idx(idx_v, dst_v, att_v, sem_i):
            pltpu.make_async_copy(
                srcN_hbm.at[pl.ds(0, K)], idx_v, sem_i).wait()
            pltpu.make_async_copy(
                dst_hbm.at[pl.ds(0, K)], dst_v, sem_i).wait()
            if use_att:
                pltpu.make_async_copy(
                    att_hbm.at[pl.ds(0, K)], att_v, sem_i).wait()

        def issue_gather(g, idx_v, rows_v, sem_r):
            pltpu.async_copy(table_hbm.at[idx_v], rows_v, sem_r)

        def wait_gather(idx_v, rows_v, sem_r):
            pltpu.make_async_copy(table_hbm.at[idx_v], rows_v, sem_r).wait()

        def issue_ew(g):
            pltpu.async_copy(
                ew_hbm.at[pl.ds(e_off + tile_base + g * K, K)], ew_s, sem_e)

        def wait_ew():
            pltpu.make_async_copy(
                ew_hbm.at[pl.ds(0, K)], ew_s, sem_e).wait()

        def drain_m(rows_v, sem_m):
            pltpu.make_async_copy(
                rows_v, m_hbm.at[pl.ds(0, K)], sem_m).wait()

        def compute_rows(rows_v, ew_v, att_v):
            def group(g2, c2):
                e0 = g2 * LANES
                if use_att:
                    att16 = att_v[pl.ds(e0, LANES)]
                for l in range(LANES):
                    e = e0 + l
                    for j in range(DH // LANES):
                        sl = pl.ds(j * LANES, LANES)
                        v = jnp.maximum(rows_v[e, sl] + ew_v[e, sl], 0.0)
                        if use_att:
                            v = v * att16[l]
                        rows_v[e, sl] = v
                return c2

            lax.fori_loop(0, K // LANES, group, 0)

        def store_scatter(g, rows_v, dst_v, sem_m):
            if store_m:
                pltpu.async_copy(
                    rows_v, m_hbm.at[pl.ds(e_off + tile_base + g * K, K)],
                    sem_m)
            pltpu.sync_copy(rows_v, acc.at[dst_v], add=True)

        A = (idx_a, dst_a, att_a, rows_a, sem_ra, sem_ma)
        B = (idx_b, dst_b, att_b, rows_b, sem_rb, sem_mb)

        issue_idx(0, A[0], A[1], A[2], A[4])
        issue_idx(1, B[0], B[1], B[2], B[4])
        issue_ew(0)
        wait_idx(A[0], A[1], A[2], A[4])
        issue_gather(0, A[0], A[3], A[4])

        def phase(g, p, P, Q, first):
            idx_p, dst_p, att_p, rows_p, sr_p, sm_p = P
            idx_q, dst_q, att_q, rows_q, sr_q, sm_q = Q
            wait_gather(idx_p, rows_p, sr_p)

            @pl.when(g + 1 < NCH)
            def _():
                wait_idx(idx_q, dst_q, att_q, sr_q)
                if store_m:
                    if first:
                        @pl.when(p > 0)
                        def _():
                            drain_m(rows_q, sm_q)
                    else:
                        drain_m(rows_q, sm_q)
                issue_gather(g + 1, idx_q, rows_q, sr_q)

            wait_ew()
            compute_rows(rows_p, ew_s, att_p)

            @pl.when(g + 1 < NCH)
            def _():
                issue_ew(g + 1)

            store_scatter(g, rows_p, dst_p, sm_p)

            @pl.when(g + 2 < NCH)
            def _():
                issue_idx(g + 2, idx_p, dst_p, att_p, sr_p)

        def pair(p, carry):
            a = 2 * p
            phase(a, p, A, B, True)

            @pl.when(a + 1 < NCH)
            def _():
                phase(a + 1, p, B, A, False)

            return carry

        lax.fori_loop(0, PAIRS, pair, 0)
        if store_m:
            drain_m(rows_a, sem_ma)
            drain_m(rows_b, sem_mb)
        plsc.subcore_barrier()

        @pl.when(sid < WT)
        def _():
            pltpu.sync_copy(
                acc.at[pl.ds(sid * RPT, RPT)],
                agg_hbm.at[pl.ds(row_off + sid * RPT, RPT)],
            )

    zrows = jnp.zeros((RPT, DH), jnp.float32)
    args = [srcN, dst, table, ew]
    if use_att:
        args.append(att)
    args.append(zrows)

    out = pl.kernel(body, out_type=out_type, mesh=mesh, scratch_types=scratch)(*args)
    return tuple(out) if store_m else out[0]


def _sc_scale_agg(m, att, dst, N, E):
    _, DH = m.shape
    EPT = E // NS
    K = _pick_chunk(EPT, mult=LANES)
    NCH = EPT // K
    WT, RPT = _pick_writers(N)
    PAIRS = (NCH + 1) // 2

    mesh = plsc.VectorSubcoreMesh(core_axis_name="c", subcore_axis_name="s")

    scratch = [
        pltpu.VMEM((K,), jnp.int32),
        pltpu.VMEM((K,), jnp.int32),
        pltpu.VMEM((K,), jnp.float32),
        pltpu.VMEM((K,), jnp.float32),
        pltpu.VMEM((K, DH), jnp.float32),
        pltpu.VMEM((K, DH), jnp.float32),
        pltpu.VMEM_SHARED((N, DH), jnp.float32),
        pltpu.SemaphoreType.DMA,
        pltpu.SemaphoreType.DMA,
    ]

    def body(m_hbm, att_hbm, dst_hbm, z_hbm, agg_hbm,
             dst_a, dst_b, att_a, att_b, rows_a, rows_b, acc,
             sem_ra, sem_rb):
        cid = lax.axis_index("c")
        sid = lax.axis_index("s")

        @pl.when(sid < WT)
        def _():
            pltpu.sync_copy(z_hbm, acc.at[pl.ds(sid * RPT, RPT)])

        plsc.subcore_barrier()
        tile_base = sid * EPT
        e_off = cid * E

        def issue(g, dst_v, att_v, rows_v, sem_r):
            pltpu.async_copy(
                m_hbm.at[pl.ds(e_off + tile_base + g * K, K)], rows_v, sem_r)
            pltpu.async_copy(
                dst_hbm.at[pl.ds(tile_base + g * K, K)], dst_v, sem_r)
            pltpu.async_copy(
                att_hbm.at[pl.ds(tile_base + g * K, K)], att_v, sem_r)

        def wait_data(dst_v, att_v, rows_v, sem_r):
            pltpu.make_async_copy(
                m_hbm.at[pl.ds(0, K)], rows_v, sem_r).wait()
            pltpu.make_async_copy(
                dst_hbm.at[pl.ds(0, K)], dst_v, sem_r).wait()
            pltpu.make_async_copy(
                att_hbm.at[pl.ds(0, K)], att_v, sem_r).wait()

        def compute_scatter(rows_v, dst_v, att_v):
            def group(g2, c2):
                e0 = g2 * LANES
                att16 = att_v[pl.ds(e0, LANES)]
                for l in range(LANES):
                    e = e0 + l
                    for j in range(DH // LANES):
                        sl = pl.ds(j * LANES, LANES)
                        rows_v[e, sl] = rows_v[e, sl] * att16[l]
                return c2

            lax.fori_loop(0, K // LANES, group, 0)
            pltpu.sync_copy(rows_v, acc.at[dst_v], add=True)

        A = (dst_a, att_a, rows_a, sem_ra)
        B = (dst_b, att_b, rows_b, sem_rb)
        issue(0, *A)
        issue(1, *B)

        def phase(g, P):
            dst_v, att_v, rows_v, sem_r = P
            wait_data(dst_v, att_v, rows_v, sem_r)
            compute_scatter(rows_v, dst_v, att_v)

            @pl.when(g + 2 < NCH)
            def _():
                issue(g + 2, dst_v, att_v, rows_v, sem_r)

        def pair(p, carry):
            a = 2 * p
            phase(a, A)

            @pl.when(a + 1 < NCH)
            def _():
                phase(a + 1, B)

            return carry

        lax.fori_loop(0, PAIRS, pair, 0)
        plsc.subcore_barrier()

        @pl.when(sid < WT)
        def _():
            pltpu.sync_copy(
                acc.at[pl.ds(sid * RPT, RPT)],
                agg_hbm.at[pl.ds(cid * N + sid * RPT, RPT)],
            )

    zrows = jnp.zeros((RPT, DH), jnp.float32)
    out = pl.kernel(
        body,
        out_type=[jax.ShapeDtypeStruct((2 * N, DH), jnp.float32)],
        mesh=mesh,
        scratch_types=scratch,
    )(m, att, dst, zrows)
    return out[0]


def _sc_att(src3, dst3, embA, embB, we2, E):
    N, D = embA.shape
    NW = NC * NS
    EPT = E // NW
    _, NCH, K = src3.shape
    PAIRS = (NCH + 1) // 2

    mesh = plsc.VectorSubcoreMesh(core_axis_name="c", subcore_axis_name="s")

    scratch = [
        pltpu.VMEM((NCH, K), jnp.int32),
        pltpu.VMEM((NCH, K), jnp.int32),
        pltpu.VMEM((K, D), jnp.float32),
        pltpu.VMEM((K, D), jnp.float32),
        pltpu.VMEM((K, D), jnp.float32),
        pltpu.VMEM((K, D), jnp.float32),
        pltpu.VMEM((K, LANES), jnp.float32),
        pltpu.VMEM((K, LANES), jnp.float32),
        pltpu.VMEM((D,), jnp.float32),
        pltpu.SemaphoreType.DMA,
        pltpu.SemaphoreType.DMA,
        pltpu.SemaphoreType.DMA,
        pltpu.SemaphoreType.DMA,
    ]

    def body(src3_hbm, dst3_hbm, a_hbm, b_hbm, w_hbm, s_hbm,
             sidx_all, didx_all, a_va, a_vb, b_va, b_vb, o_va, o_vb, w_v,
             sem_a, sem_b, sem_oa, sem_ob):
        cid = lax.axis_index("c")
        sid = lax.axis_index("s")
        wid = sid * NC + cid
        pltpu.sync_copy(w_hbm, w_v)
        pltpu.sync_copy(src3_hbm.at[wid], sidx_all)
        pltpu.sync_copy(dst3_hbm.at[wid], didx_all)
        tile_base = wid * EPT

        def issue(g, a_v, b_v, sem):
            pltpu.async_copy(a_hbm.at[sidx_all.at[g]], a_v, sem)
            pltpu.async_copy(b_hbm.at[didx_all.at[g]], b_v, sem)

        def wait_data(a_v, b_v, sem):
            pltpu.make_async_copy(a_hbm.at[sidx_all.at[0]], a_v, sem).wait()
            pltpu.make_async_copy(b_hbm.at[didx_all.at[0]], b_v, sem).wait()

        def drain_out(o_v, sem_o):
            pltpu.make_async_copy(o_v, s_hbm.at[pl.ds(0, K)], sem_o).wait()

        def compute_store(g, a_v, b_v, o_v, sem_o):
            def edge(e, c2):
                acc = jnp.zeros((LANES,), jnp.float32)
                for j in range(D // LANES):
                    sl = pl.ds(j * LANES, LANES)
                    t = jnp.maximum(a_v[e, sl] + b_v[e, sl], 0.0)
                    acc = acc + t * w_v[pl.ds(j * LANES, LANES)]
                o_v[e, pl.ds(0, LANES)] = acc
                return c2

            lax.fori_loop(0, K, edge, 0)
            pltpu.async_copy(o_v, s_hbm.at[pl.ds(tile_base + g * K, K)], sem_o)

        issue(0, a_va, b_va, sem_a)
        issue(1, a_vb, b_vb, sem_b)

        def pair(p, carry):
            a = 2 * p
            b = a + 1
            wait_data(a_va, b_va, sem_a)
            compute_store(a, a_va, b_va, o_va, sem_oa)

            @pl.when(a + 2 < NCH)
            def _():
                drain_out(o_va, sem_oa)
                issue(a + 2, a_va, b_va, sem_a)

            @pl.when(b < NCH)
            def _():
                wait_data(a_vb, b_vb, sem_b)
                compute_store(b, a_vb, b_vb, o_vb, sem_ob)

                @pl.when(b + 2 < NCH)
                def _():
                    drain_out(o_vb, sem_ob)
                    issue(b + 2, a_vb, b_vb, sem_b)

            return carry

        lax.fori_loop(0, PAIRS, pair, 0)
        drain_out(o_va, sem_oa)
        drain_out(o_vb, sem_ob)

    out = pl.kernel(
        body,
        out_type=[jax.ShapeDtypeStruct((E, LANES), jnp.float32)],
        mesh=mesh,
        scratch_types=scratch,
    )(src3, dst3, embA, embB, we2)
    return out[0]


def kernel(x, edge_index, edge_attr, batch, W1n, W1e, W1s, b1,
           W2n, W2e, W2s, b2, We1, be1, We2, be2):
    N, D = x.shape
    E = edge_index.shape[1]
    src = edge_index[0]
    dst = edge_index[1]
    b1r = b1.reshape(1, D)
    b2r = b2.reshape(1, D)
    be1r = be1.reshape(1, D)
    we2v = We2.reshape(D)
    be2r = be2.reshape(1, 1)

    EPT = E // NS
    K = _pick_chunk(EPT, mult=LANES)
    NCH = EPT // K
    srcN = jnp.concatenate([src, src + N], axis=0)

    NW = NC * NS
    EPW = E // NW
    KA = _pick_chunk(EPW, 64)
    NCHA = EPW // KA
    srcA = src.reshape(NW, NCHA, KA)
    dstA = dst.reshape(NW, NCHA, KA)

    xW1n_t, xW1s = _tc_pre_node(x, W1n, W1s)
    eW1_t, eW2_t = _tc_pre_edge(edge_attr, W1e, W2e)

    m1, agg1 = _sc_msgpass(srcN, dst, xW1n_t, eW1_t, None, True, N, E)
    h1W2n_t, h1W2s = _tc_mid(agg1, xW1s, b1r, W2n, W2s)
    agg2 = _sc_msgpass(srcN, dst, h1W2n_t, eW2_t, None, False, N, E)
    embA, embB = _tc_emb(agg2, h1W2s, b2r, We1, be1r)
    s16 = _sc_att(srcA, dstA, embA, embB, we2v, E)
    logits, att2 = _tc_logits(s16, be2r)
    attf = att2.reshape(E)

    agg1p = _sc_scale_agg(m1, attf, dst, N, E)
    h1pW2n_t, h1pW2s = _tc_mid(agg1p, xW1s, b1r, W2n, W2s)
    agg2p = _sc_msgpass(srcN, dst, h1pW2n_t, eW2_t, attf, False, N, E)
    node_embeddings = _tc_final(agg2p, h1pW2s, b2r)

    return (logits, att2, node_embeddings)


# speedup vs baseline: 1.8320x; 1.0874x over previous
import functools

import jax
import jax.numpy as jnp
from jax import lax
from jax.experimental import pallas as pl
from jax.experimental.pallas import tpu as pltpu
from jax.experimental.pallas import tpu_sc as plsc

NC = 2
NS = 16
LANES = 16


def _pick_row_block(n, target=1024):
    for r in range(min(n, target), 7, -8):
        if n % r == 0:
            return r
    return n


def _dot(a, b):
    return jnp.dot(a, b, preferred_element_type=jnp.float32)


def _tc_pre_node(x, W1n, W1s):
    N, D = x.shape
    DH = D // 2
    R = _pick_row_block(N)
    NB = N // R

    def body(x_ref, w1n_ref, w1s_ref, t_ref, s_ref):
        a = x_ref[...]
        t_ref[...] = _dot(a, w1n_ref[...])
        s_ref[...] = _dot(a, w1s_ref[...])

    return pl.pallas_call(
        body,
        grid=(NB, 2),
        in_specs=[
            pl.BlockSpec((R, D), lambda i, j: (i, 0)),
            pl.BlockSpec((D, DH), lambda i, j: (0, j)),
            pl.BlockSpec((D, DH), lambda i, j: (0, j)),
        ],
        out_specs=[
            pl.BlockSpec((R, DH), lambda i, j: (j * NB + i, 0)),
            pl.BlockSpec((R, DH), lambda i, j: (i, j)),
        ],
        out_shape=[
            jax.ShapeDtypeStruct((2 * N, DH), jnp.float32),
            jax.ShapeDtypeStruct((N, D), jnp.float32),
        ],
    )(x, W1n, W1s)


def _tc_pre_edge(ea, W1e, W2e):
    E, DE = ea.shape
    D = W1e.shape[1]
    DH = D // 2
    R = _pick_row_block(E, 8000)
    EB = E // R

    def body(ea_ref, w1_ref, w2_ref, o1_ref, o2_ref):
        a = ea_ref[...]
        o1_ref[...] = _dot(a, w1_ref[...])
        o2_ref[...] = _dot(a, w2_ref[...])

    return pl.pallas_call(
        body,
        grid=(EB, 2),
        in_specs=[
            pl.BlockSpec((R, DE), lambda i, j: (i, 0)),
            pl.BlockSpec((DE, DH), lambda i, j: (0, j)),
            pl.BlockSpec((DE, DH), lambda i, j: (0, j)),
        ],
        out_specs=[
            pl.BlockSpec((R, DH), lambda i, j: (j * EB + i, 0)),
            pl.BlockSpec((R, DH), lambda i, j: (j * EB + i, 0)),
        ],
        out_shape=[
            jax.ShapeDtypeStruct((2 * E, DH), jnp.float32),
            jax.ShapeDtypeStruct((2 * E, DH), jnp.float32),
        ],
    )(ea, W1e, W2e)


def _tc_mid(agg, skip, b, Wn, Ws):
    N, D = skip.shape
    DH = D // 2
    R = _pick_row_block(N)
    NB = N // R

    def body(lo_ref, hi_ref, skip_ref, b_ref, wn_ref, ws_ref, t_ref, s_ref):
        h = jnp.concatenate([lo_ref[...], hi_ref[...]], axis=1)
        h = jnp.maximum(h + skip_ref[...] + b_ref[...], 0.0)
        t_ref[...] = _dot(h, wn_ref[...])
        s_ref[...] = _dot(h, ws_ref[...])

    return pl.pallas_call(
        body,
        grid=(NB, 2),
        in_specs=[
            pl.BlockSpec((R, DH), lambda i, j: (i, 0)),
            pl.BlockSpec((R, DH), lambda i, j: (NB + i, 0)),
            pl.BlockSpec((R, D), lambda i, j: (i, 0)),
            pl.BlockSpec((1, D), lambda i, j: (0, 0)),
            pl.BlockSpec((D, DH), lambda i, j: (0, j)),
            pl.BlockSpec((D, DH), lambda i, j: (0, j)),
        ],
        out_specs=[
            pl.BlockSpec((R, DH), lambda i, j: (j * NB + i, 0)),
            pl.BlockSpec((R, DH), lambda i, j: (i, j)),
        ],
        out_shape=[
            jax.ShapeDtypeStruct((2 * N, DH), jnp.float32),
            jax.ShapeDtypeStruct((N, D), jnp.float32),
        ],
    )(agg, agg, skip, b, Wn, Ws)


def _tc_emb(agg, skip, b, We1, be1):
    N, D = skip.shape
    DH = D // 2
    R = _pick_row_block(N)
    NB = N // R

    def body(lo_ref, hi_ref, skip_ref, b_ref, wa_ref, wb_ref, be1_ref, a_ref, b2_ref):
        h = jnp.concatenate([lo_ref[...], hi_ref[...]], axis=1)
        h = jnp.maximum(h + skip_ref[...] + b_ref[...], 0.0)
        a_ref[...] = _dot(h, wa_ref[...]) + be1_ref[...]
        b2_ref[...] = _dot(h, wb_ref[...])

    return pl.pallas_call(
        body,
        grid=(NB, 2),
        in_specs=[
            pl.BlockSpec((R, DH), lambda i, j: (i, 0)),
            pl.BlockSpec((R, DH), lambda i, j: (NB + i, 0)),
            pl.BlockSpec((R, D), lambda i, j: (i, 0)),
            pl.BlockSpec((1, D), lambda i, j: (0, 0)),
            pl.BlockSpec((D, DH), lambda i, j: (0, j)),
            pl.BlockSpec((D, DH), lambda i, j: (1, j)),
            pl.BlockSpec((1, DH), lambda i, j: (0, j)),
        ],
        out_specs=[
            pl.BlockSpec((R, DH), lambda i, j: (i, j)),
            pl.BlockSpec((R, DH), lambda i, j: (i, j)),
        ],
        out_shape=[
            jax.ShapeDtypeStruct((N, D), jnp.float32),
            jax.ShapeDtypeStruct((N, D), jnp.float32),
        ],
    )(agg, agg, skip, b, We1, We1, be1)


def _tc_final(agg, skip, b):
    N, D = skip.shape
    DH = D // 2
    R = _pick_row_block(N)
    NB = N // R

    def body(agg_ref, skip_ref, b_ref, o_ref):
        o_ref[...] = jnp.maximum(agg_ref[...] + skip_ref[...] + b_ref[...], 0.0)

    return pl.pallas_call(
        body,
        grid=(NB, 2),
        in_specs=[
            pl.BlockSpec((R, DH), lambda i, j: (j * NB + i, 0)),
            pl.BlockSpec((R, DH), lambda i, j: (i, j)),
            pl.BlockSpec((1, DH), lambda i, j: (0, j)),
        ],
        out_specs=pl.BlockSpec((R, DH), lambda i, j: (i, j)),
        out_shape=jax.ShapeDtypeStruct((N, D), jnp.float32),
    )(agg, skip, b)


def _tc_logits(s16, be2):
    E, L = s16.shape
    R = _pick_row_block(E, 8000)
    EB = E // R

    def body(s_ref, b_ref, lo_ref, at_ref):
        v = jnp.sum(s_ref[...], axis=1, keepdims=True) + b_ref[...]
        lo_ref[...] = v
        at_ref[...] = jax.nn.sigmoid(v)

    return pl.pallas_call(
        body,
        grid=(EB,),
        in_specs=[
            pl.BlockSpec((R, L), lambda i: (i, 0)),
            pl.BlockSpec((1, 1), lambda i: (0, 0)),
        ],
        out_specs=[
            pl.BlockSpec((R, 1), lambda i: (i, 0)),
            pl.BlockSpec((R, 1), lambda i: (i, 0)),
        ],
        out_shape=[
            jax.ShapeDtypeStruct((E, 1), jnp.float32),
            jax.ShapeDtypeStruct((E, 1), jnp.float32),
        ],
    )(s16, be2)


def _pick_chunk(n, cap=128, mult=8):
    for k in range(cap - cap % mult, mult - 1, -mult):
        if n % k == 0:
            return k
    return mult


def _pick_writers(n):
    for wt in range(NS, 0, -1):
        if n % wt == 0 and (n // wt) % 8 == 0:
            return wt, n // wt
    return 1, n


def _sc_msgpass(srcN, dst, table, ew, att, store_m, N, E):
    _, DH = table.shape
    EPT = E // NS
    K = _pick_chunk(EPT, mult=LANES)
    NCH = EPT // K
    WT, RPT = _pick_writers(N)
    use_att = att is not None
    PAIRS = (NCH + 1) // 2

    mesh = plsc.VectorSubcoreMesh(core_axis_name="c", subcore_axis_name="s")

    out_type = [jax.ShapeDtypeStruct((2 * N, DH), jnp.float32)]
    if store_m:
        out_type = [jax.ShapeDtypeStruct((2 * E, DH), jnp.float32)] + out_type

    scratch = [
        pltpu.VMEM((K,), jnp.int32),
        pltpu.VMEM((K,), jnp.int32),
        pltpu.VMEM((K,), jnp.int32),
        pltpu.VMEM((K,), jnp.int32),
        pltpu.VMEM((K,), jnp.float32),
        pltpu.VMEM((K,), jnp.float32),
        pltpu.VMEM((K, DH), jnp.float32),
        pltpu.VMEM((K, DH), jnp.float32),
        pltpu.VMEM((K, DH), jnp.float32),
        pltpu.VMEM_SHARED((N, DH), jnp.float32),
        pltpu.SemaphoreType.DMA,
        pltpu.SemaphoreType.DMA,
        pltpu.SemaphoreType.DMA,
        pltpu.SemaphoreType.DMA,
        pltpu.SemaphoreType.DMA,
    ]

    def body(*refs):
        i = 0
        srcN_hbm = refs[i]; i += 1
        dst_hbm = refs[i]; i += 1
        table_hbm = refs[i]; i += 1
        ew_hbm = refs[i]; i += 1
        if use_att:
            att_hbm = refs[i]; i += 1
        z_hbm = refs[i]; i += 1
        if store_m:
            m_hbm = refs[i]; i += 1
        agg_hbm = refs[i]; i += 1
        (idx_a, idx_b, dst_a, dst_b, att_a, att_b,
         rows_a, rows_b, ew_s, acc,
         sem_ra, sem_rb, sem_e, sem_ma, sem_mb) = refs[i:]

        cid = lax.axis_index("c")
        sid = lax.axis_index("s")

        @pl.when(sid < WT)
        def _():
            pltpu.sync_copy(z_hbm, acc.at[pl.ds(sid * RPT, RPT)])

        plsc.subcore_barrier()

        tile_base = sid * EPT
        row_off = cid * N
        e_off = cid * E

        def issue_idx(g, idx_v, dst_v, att_v, sem_i):
            pltpu.async_copy(
                srcN_hbm.at[pl.ds(e_off + tile_base + g * K, K)], idx_v, sem_i)
            pltpu.async_copy(
                dst_hbm.at[pl.ds(tile_base + g * K, K)], dst_v, sem_i)
            if use_att:
                pltpu.async_copy(
                    att_hbm.at[pl.ds(tile_base + g * K, K)], att_v, sem_i)

        def wait_idx(idx_v, dst_v, att_v, sem_i):
            pltpu.make_async_copy(
                srcN_hbm.at[pl.ds(0, K)], idx_v, sem_i).wait()
            pltpu.make_async_copy(
                dst_hbm.at[pl.ds(0, K)], dst_v, sem_i).wait()
            if use_att:
                pltpu.make_async_copy(
                    att_hbm.at[pl.ds(0, K)], att_v, sem_i).wait()

        def issue_gather(g, idx_v, rows_v, sem_r):
            pltpu.async_copy(table_hbm.at[idx_v], rows_v, sem_r)

        def wait_gather(idx_v, rows_v, sem_r):
            pltpu.make_async_copy(table_hbm.at[idx_v], rows_v, sem_r).wait()

        def issue_ew(g):
            pltpu.async_copy(
                ew_hbm.at[pl.ds(e_off + tile_base + g * K, K)], ew_s, sem_e)

        def wait_ew():
            pltpu.make_async_copy(
                ew_hbm.at[pl.ds(0, K)], ew_s, sem_e).wait()

        def drain_m(rows_v, sem_m):
            pltpu.make_async_copy(
                rows_v, m_hbm.at[pl.ds(0, K)], sem_m).wait()

        def compute_rows(rows_v, ew_v, att_v):
            def group(g2, c2):
                e0 = g2 * LANES
                if use_att:
                    att16 = att_v[pl.ds(e0, LANES)]
                for l in range(LANES):
                    e = e0 + l
                    for j in range(DH // LANES):
                        sl = pl.ds(j * LANES, LANES)
                        v = jnp.maximum(rows_v[e, sl] + ew_v[e, sl], 0.0)
                        if use_att:
                            v = v * att16[l]
                        rows_v[e, sl] = v
                return c2

            lax.fori_loop(0, K // LANES, group, 0)

        def store_scatter(g, rows_v, dst_v, sem_m):
            if store_m:
                pltpu.async_copy(
                    rows_v, m_hbm.at[pl.ds(e_off + tile_base + g * K, K)],
                    sem_m)
            pltpu.sync_copy(rows_v, acc.at[dst_v], add=True)

        A = (idx_a, dst_a, att_a, rows_a, sem_ra, sem_ma)
        B = (idx_b, dst_b, att_b, rows_b, sem_rb, sem_mb)

        issue_idx(0, A[0], A[1], A[2], A[4])
        issue_idx(1, B[0], B[1], B[2], B[4])
        issue_ew(0)
        wait_idx(A[0], A[1], A[2], A[4])
        issue_gather(0, A[0], A[3], A[4])

        def phase(g, p, P, Q, first):
            idx_p, dst_p, att_p, rows_p, sr_p, sm_p = P
            idx_q, dst_q, att_q, rows_q, sr_q, sm_q = Q
            wait_gather(idx_p, rows_p, sr_p)

            @pl.when(g + 1 < NCH)
            def _():
                wait_idx(idx_q, dst_q, att_q, sr_q)
                if store_m:
                    if first:
                        @pl.when(p > 0)
                        def _():
                            drain_m(rows_q, sm_q)
                    else:
                        drain_m(rows_q, sm_q)
                issue_gather(g + 1, idx_q, rows_q, sr_q)

            wait_ew()
            compute_rows(rows_p, ew_s, att_p)

            @pl.when(g + 1 < NCH)
            def _():
                issue_ew(g + 1)

            store_scatter(g, rows_p, dst_p, sm_p)

            @pl.when(g + 2 < NCH)
            def _():
                issue_idx(g + 2, idx_p, dst_p, att_p, sr_p)

        def pair(p, carry):
            a = 2 * p
            phase(a, p, A, B, True)

            @pl.when(a + 1 < NCH)
            def _():
                phase(a + 1, p, B, A, False)

            return carry

        lax.fori_loop(0, PAIRS, pair, 0)
        if store_m:
            drain_m(rows_a, sem_ma)
            drain_m(rows_b, sem_mb)
        plsc.subcore_barrier()

        @pl.when(sid < WT)
        def _():
            pltpu.sync_copy(
                acc.at[pl.ds(sid * RPT, RPT)],
                agg_hbm.at[pl.ds(row_off + sid * RPT, RPT)],
            )

    zrows = jnp.zeros((RPT, DH), jnp.float32)
    args = [srcN, dst, table, ew]
    if use_att:
        args.append(att)
    args.append(zrows)

    out = pl.kernel(body, out_type=out_type, mesh=mesh, scratch_types=scratch)(*args)
    return tuple(out) if store_m else out[0]


def _sc_scale_agg(m, att, dst, N, E):
    _, DH = m.shape
    EPT = E // NS
    K = _pick_chunk(EPT, mult=LANES)
    NCH = EPT // K
    WT, RPT = _pick_writers(N)
    PAIRS = (NCH + 1) // 2

    mesh = plsc.VectorSubcoreMesh(core_axis_name="c", subcore_axis_name="s")

    scratch = [
        pltpu.VMEM((K,), jnp.int32),
        pltpu.VMEM((K,), jnp.int32),
        pltpu.VMEM((K,), jnp.float32),
        pltpu.VMEM((K,), jnp.float32),
        pltpu.VMEM((K, DH), jnp.float32),
        pltpu.VMEM((K, DH), jnp.float32),
        pltpu.VMEM_SHARED((N, DH), jnp.float32),
        pltpu.SemaphoreType.DMA,
        pltpu.SemaphoreType.DMA,
    ]

    def body(m_hbm, att_hbm, dst_hbm, z_hbm, agg_hbm,
             dst_a, dst_b, att_a, att_b, rows_a, rows_b, acc,
             sem_ra, sem_rb):
        cid = lax.axis_index("c")
        sid = lax.axis_index("s")

        @pl.when(sid < WT)
        def _():
            pltpu.sync_copy(z_hbm, acc.at[pl.ds(sid * RPT, RPT)])

        plsc.subcore_barrier()
        tile_base = sid * EPT
        e_off = cid * E

        def issue(g, dst_v, att_v, rows_v, sem_r):
            pltpu.async_copy(
                m_hbm.at[pl.ds(e_off + tile_base + g * K, K)], rows_v, sem_r)
            pltpu.async_copy(
                dst_hbm.at[pl.ds(tile_base + g * K, K)], dst_v, sem_r)
            pltpu.async_copy(
                att_hbm.at[pl.ds(tile_base + g * K, K)], att_v, sem_r)

        def wait_data(dst_v, att_v, rows_v, sem_r):
            pltpu.make_async_copy(
                m_hbm.at[pl.ds(0, K)], rows_v, sem_r).wait()
            pltpu.make_async_copy(
                dst_hbm.at[pl.ds(0, K)], dst_v, sem_r).wait()
            pltpu.make_async_copy(
                att_hbm.at[pl.ds(0, K)], att_v, sem_r).wait()

        def compute_scatter(rows_v, dst_v, att_v):
            def group(g2, c2):
                e0 = g2 * LANES
                att16 = att_v[pl.ds(e0, LANES)]
                for l in range(LANES):
                    e = e0 + l
                    for j in range(DH // LANES):
                        sl = pl.ds(j * LANES, LANES)
                        rows_v[e, sl] = rows_v[e, sl] * att16[l]
                return c2

            lax.fori_loop(0, K // LANES, group, 0)
            pltpu.sync_copy(rows_v, acc.at[dst_v], add=True)

        A = (dst_a, att_a, rows_a, sem_ra)
        B = (dst_b, att_b, rows_b, sem_rb)
        issue(0, *A)
        issue(1, *B)

        def phase(g, P):
            dst_v, att_v, rows_v, sem_r = P
            wait_data(dst_v, att_v, rows_v, sem_r)
            compute_scatter(rows_v, dst_v, att_v)

            @pl.when(g + 2 < NCH)
            def _():
                issue(g + 2, dst_v, att_v, rows_v, sem_r)

        def pair(p, carry):
            a = 2 * p
            phase(a, A)

            @pl.when(a + 1 < NCH)
            def _():
                phase(a + 1, B)

            return carry

        lax.fori_loop(0, PAIRS, pair, 0)
        plsc.subcore_barrier()

        @pl.when(sid < WT)
        def _():
            pltpu.sync_copy(
                acc.at[pl.ds(sid * RPT, RPT)],
                agg_hbm.at[pl.ds(cid * N + sid * RPT, RPT)],
            )

    zrows = jnp.zeros((RPT, DH), jnp.float32)
    out = pl.kernel(
        body,
        out_type=[jax.ShapeDtypeStruct((2 * N, DH), jnp.float32)],
        mesh=mesh,
        scratch_types=scratch,
    )(m, att, dst, zrows)
    return out[0]


def _sc_att(src3, dst3, embA, embB, we2, E):
    N, D = embA.shape
    NW = NC * NS
    EPT = E // NW
    _, NCH, K = src3.shape
    PAIRS = (NCH + 1) // 2

    mesh = plsc.VectorSubcoreMesh(core_axis_name="c", subcore_axis_name="s")

    scratch = [
        pltpu.VMEM((NCH, K), jnp.int32),
        pltpu.VMEM((NCH, K), jnp.int32),
        pltpu.VMEM((K, D), jnp.float32),
        pltpu.VMEM((K, D), jnp.float32),
        pltpu.VMEM((K, D), jnp.float32),
        pltpu.VMEM((K, D), jnp.float32),
        pltpu.VMEM((K, LANES), jnp.float32),
        pltpu.VMEM((K, LANES), jnp.float32),
        pltpu.VMEM((D,), jnp.float32),
        pltpu.SemaphoreType.DMA,
        pltpu.SemaphoreType.DMA,
        pltpu.SemaphoreType.DMA,
        pltpu.SemaphoreType.DMA,
    ]

    def body(src3_hbm, dst3_hbm, a_hbm, b_hbm, w_hbm, s_hbm,
             sidx_all, didx_all, a_va, a_vb, b_va, b_vb, o_va, o_vb, w_v,
             sem_a, sem_b, sem_oa, sem_ob):
        cid = lax.axis_index("c")
        sid = lax.axis_index("s")
        wid = sid * NC + cid
        pltpu.sync_copy(w_hbm, w_v)
        pltpu.sync_copy(src3_hbm.at[wid], sidx_all)
        pltpu.sync_copy(dst3_hbm.at[wid], didx_all)
        tile_base = wid * EPT

        def issue(g, a_v, b_v, sem):
            pltpu.async_copy(a_hbm.at[sidx_all.at[g]], a_v, sem)
            pltpu.async_copy(b_hbm.at[didx_all.at[g]], b_v, sem)

        def wait_data(a_v, b_v, sem):
            pltpu.make_async_copy(a_hbm.at[sidx_all.at[0]], a_v, sem).wait()
            pltpu.make_async_copy(b_hbm.at[didx_all.at[0]], b_v, sem).wait()

        def drain_out(o_v, sem_o):
            pltpu.make_async_copy(o_v, s_hbm.at[pl.ds(0, K)], sem_o).wait()

        def compute_store(g, a_v, b_v, o_v, sem_o):
            def edge(e, c2):
                acc = jnp.zeros((LANES,), jnp.float32)
                for j in range(D // LANES):
                    sl = pl.ds(j * LANES, LANES)
                    t = jnp.maximum(a_v[e, sl] + b_v[e, sl], 0.0)
                    u = jax.lax.bitcast_convert_type(t, jnp.int32)
                    u = (u + 0x7FFF + ((u >> 16) & 1)) & ~0xFFFF
                    t = jax.lax.bitcast_convert_type(u, jnp.float32)
                    acc = acc + t * w_v[pl.ds(j * LANES, LANES)]
                o_v[e, pl.ds(0, LANES)] = acc
                return c2

            lax.fori_loop(0, K, edge, 0)
            pltpu.async_copy(o_v, s_hbm.at[pl.ds(tile_base + g * K, K)], sem_o)

        issue(0, a_va, b_va, sem_a)
        issue(1, a_vb, b_vb, sem_b)

        def pair(p, carry):
            a = 2 * p
            b = a + 1
            wait_data(a_va, b_va, sem_a)
            compute_store(a, a_va, b_va, o_va, sem_oa)

            @pl.when(a + 2 < NCH)
            def _():
                drain_out(o_va, sem_oa)
                issue(a + 2, a_va, b_va, sem_a)

            @pl.when(b < NCH)
            def _():
                wait_data(a_vb, b_vb, sem_b)
                compute_store(b, a_vb, b_vb, o_vb, sem_ob)

                @pl.when(b + 2 < NCH)
                def _():
                    drain_out(o_vb, sem_ob)
                    issue(b + 2, a_vb, b_vb, sem_b)

            return carry

        lax.fori_loop(0, PAIRS, pair, 0)
        drain_out(o_va, sem_oa)
        drain_out(o_vb, sem_ob)

    out = pl.kernel(
        body,
        out_type=[jax.ShapeDtypeStruct((E, LANES), jnp.float32)],
        mesh=mesh,
        scratch_types=scratch,
    )(src3, dst3, embA, embB, we2)
    return out[0]


def kernel(x, edge_index, edge_attr, batch, W1n, W1e, W1s, b1,
           W2n, W2e, W2s, b2, We1, be1, We2, be2):
    N, D = x.shape
    E = edge_index.shape[1]
    src = edge_index[0]
    dst = edge_index[1]
    b1r = b1.reshape(1, D)
    b2r = b2.reshape(1, D)
    be1r = be1.reshape(1, D)
    we2v = We2.reshape(D).astype(jnp.bfloat16).astype(jnp.float32)
    be2r = be2.reshape(1, 1)

    EPT = E // NS
    K = _pick_chunk(EPT, mult=LANES)
    NCH = EPT // K
    srcN = jnp.concatenate([src, src + N], axis=0)

    NW = NC * NS
    EPW = E // NW
    KA = _pick_chunk(EPW, 64)
    NCHA = EPW // KA
    srcA = src.reshape(NW, NCHA, KA)
    dstA = dst.reshape(NW, NCHA, KA)

    xW1n_t, xW1s = _tc_pre_node(x, W1n, W1s)
    eW1_t, eW2_t = _tc_pre_edge(edge_attr, W1e, W2e)

    m1, agg1 = _sc_msgpass(srcN, dst, xW1n_t, eW1_t, None, True, N, E)
    h1W2n_t, h1W2s = _tc_mid(agg1, xW1s, b1r, W2n, W2s)
    agg2 = _sc_msgpass(srcN, dst, h1W2n_t, eW2_t, None, False, N, E)
    embA, embB = _tc_emb(agg2, h1W2s, b2r, We1, be1r)
    s16 = _sc_att(srcA, dstA, embA, embB, we2v, E)
    logits, att2 = _tc_logits(s16, be2r)
    attf = att2.reshape(E)

    agg1p = _sc_scale_agg(m1, attf, dst, N, E)
    h1pW2n_t, h1pW2s = _tc_mid(agg1p, xW1s, b1r, W2n, W2s)
    agg2p = _sc_msgpass(srcN, dst, h1pW2n_t, eW2_t, attf, False, N, E)
    node_embeddings = _tc_final(agg2p, h1pW2s, b2r)

    return (logits, att2, node_embeddings)
